# Initial kernel scaffold; baseline (speedup 1.0000x reference)
import jax
import jax.numpy as jnp
from jax.experimental import pallas as pl


def kernel(x, edge_index_r0, edge_index_r1, edge_index_r2, basis, coeff, loop_weight, h_bias):
    raise NotImplementedError("write your pallas kernel here")


# speedup vs baseline: 1.6916x; 1.6916x over previous
import jax
import jax.numpy as jnp
from jax import lax
from jax.experimental import pallas as pl
from jax.experimental.pallas import tpu as pltpu
from jax.experimental.pallas import tpu_sc as plsc

N = 50000
E = 160000
D = 128
R = 3
NB = 2

CW = 32
NCH = D // CW
NSUB = 16
ROWS_PER_TILE = 3200
N_PAD = NSUB * ROWS_PER_TILE
TRASH = N_PAD - 1

IDXW = 128
K = 4
E_PAD = 163840
EROWS = E_PAD // IDXW
TILE_EROWS = EROWS // NSUB
GROUPS = TILE_EROWS // K
ZROWS = 128

BN = 2000
GRID = N // BN

ITEMS = (
    ('f', 0, 0, 0), ('f', 0, 1, 0), ('f', 0, 2, 1), ('f', 0, 3, 1),
    ('f', 1, 0, 0), ('f', 1, 1, 0), ('f', 1, 2, 1), ('f', 1, 3, 1),
    ('f', 2, 0, 0), ('f', 2, 1, 0), ('f', 2, 2, 1), ('f', 2, 3, 1),
    ('d', 0, 0, 0), ('d', 1, 0, 1), ('d', 2, 0, 1),
)


def _tc1_body(x_ref, basis_ref, coeff_ref, loop_ref, *out_refs):
    x = x_ref[...]
    y0 = jnp.dot(x, basis_ref[0], preferred_element_type=jnp.float32)
    y1 = jnp.dot(x, basis_ref[1], preferred_element_type=jnp.float32)
    for r in range(R):
        yr = y0 * coeff_ref[r:r + 1, 0:1] + y1 * coeff_ref[r:r + 1, 1:2]
        for c in range(NCH):
            out_refs[r * NCH + c][...] = yr[:, c * CW:(c + 1) * CW]
    out_refs[R * NCH][...] = jnp.dot(x, loop_ref[...],
                                     preferred_element_type=jnp.float32)


def _tc1(x, basis, coeff, loop_weight):
    outs = [jax.ShapeDtypeStruct((N, CW), jnp.float32) for _ in range(R * NCH)]
    outs.append(jax.ShapeDtypeStruct((N, D), jnp.float32))
    out_specs = [pl.BlockSpec((BN, CW), lambda i: (i, 0)) for _ in range(R * NCH)]
    out_specs.append(pl.BlockSpec((BN, D), lambda i: (i, 0)))
    return pl.pallas_call(
        _tc1_body,
        grid=(GRID,),
        in_specs=[
            pl.BlockSpec((BN, D), lambda i: (i, 0)),
            pl.BlockSpec((NB, D, D), lambda i: (0, 0, 0)),
            pl.BlockSpec((R, NB), lambda i: (0, 0)),
            pl.BlockSpec((D, D), lambda i: (0, 0)),
        ],
        out_specs=out_specs,
        out_shape=outs,
    )(x, basis, coeff, loop_weight)


def _sc_body(*refs):
    tables = refs[0:12]
    edges = refs[12:18]
    ones_hbm = refs[18]
    zeros_hbm = refs[19]
    outs = refs[20:35]
    acc, src_buf, dst_buf, rows_v, zbuf, ones_v, gsem = refs[35:42]

    core = lax.axis_index("c")
    tid = lax.axis_index("s")

    pltpu.sync_copy(zeros_hbm, zbuf)
    pltpu.sync_copy(ones_hbm, ones_v)

    def run_item(table, src_hbm, dst_hbm, out_ref):
        def zero_body(j, carry):
            pltpu.sync_copy(
                zbuf, acc.at[pl.ds(tid * ROWS_PER_TILE + j * ZROWS, ZROWS)])
            return carry

        lax.fori_loop(0, ROWS_PER_TILE // ZROWS, zero_body, 0)
        plsc.subcore_barrier()

        def group_body(g, carry):
            base = tid * TILE_EROWS + g * K
            pltpu.sync_copy(dst_hbm.at[pl.ds(base, K)], dst_buf)
            if table is not None:
                pltpu.sync_copy(src_hbm.at[pl.ds(base, K)], src_buf)
                copies = [
                    pltpu.async_copy(table.at[src_buf.at[j]],
                                     rows_v.at[pl.ds(j * IDXW, IDXW)], gsem)
                    for j in range(K)
                ]
                for cp in copies:
                    cp.wait()
                for j in range(K):
                    pltpu.sync_copy(rows_v.at[pl.ds(j * IDXW, IDXW)],
                                    acc.at[dst_buf.at[j]], add=True)
            else:
                for j in range(K):
                    pltpu.sync_copy(ones_v, acc.at[dst_buf.at[j]], add=True)
            return carry

        lax.fori_loop(0, GROUPS, group_body, 0)
        plsc.subcore_barrier()
        pltpu.sync_copy(acc.at[pl.ds(tid * ROWS_PER_TILE, ROWS_PER_TILE)],
                        out_ref.at[pl.ds(tid * ROWS_PER_TILE, ROWS_PER_TILE)])

    for kind, r, c, owner in ITEMS:
        if kind == 'f':
            table = tables[r * NCH + c]
            out_ref = outs[r * NCH + c]
        else:
            table = None
            out_ref = outs[R * NCH + r]

        @pl.when(core == owner)
        def _(table=table, r=r, out_ref=out_ref):
            run_item(table, edges[2 * r], edges[2 * r + 1], out_ref)


def _sc(tables, edge_pairs, ones_h, zeros_h):
    mesh = plsc.VectorSubcoreMesh(core_axis_name="c", subcore_axis_name="s")
    out_type = [jax.ShapeDtypeStruct((N_PAD, CW), jnp.float32)
                for _ in range(R * NCH + R)]
    kern = pl.kernel(
        _sc_body,
        out_type=out_type,
        mesh=mesh,
        scratch_types=[
            pltpu.VMEM_SHARED((N_PAD, CW), jnp.float32),
            pltpu.VMEM((K, IDXW), jnp.int32),
            pltpu.VMEM((K, IDXW), jnp.int32),
            pltpu.VMEM((K * IDXW, CW), jnp.float32),
            pltpu.VMEM((ZROWS, CW), jnp.float32),
            pltpu.VMEM((IDXW, CW), jnp.float32),
            pltpu.SemaphoreType.DMA,
        ],
        compiler_params=pltpu.CompilerParams(use_tc_tiling_on_sc=False),
    )
    return kern(*tables, *edge_pairs, ones_h, zeros_h)


def _tc2_body(*refs):
    accs = refs[0:12]
    degs = refs[12:15]
    xl_ref = refs[15]
    bias_ref = refs[16]
    out_ref = refs[17]
    invs = [1.0 / jnp.maximum(degs[r][...], 1.0) for r in range(R)]
    xl = xl_ref[...]
    bias = bias_ref[...]
    for c in range(NCH):
        h = xl[:, c * CW:(c + 1) * CW] + bias[:, c * CW:(c + 1) * CW]
        for r in range(R):
            h = h + accs[r * NCH + c][...] * invs[r]
        out_ref[:, c * CW:(c + 1) * CW] = h


def _tc2(accs, degs, xl, bias2d):
    in_specs = (
        [pl.BlockSpec((BN, CW), lambda i: (i, 0)) for _ in range(R * NCH + R)]
        + [pl.BlockSpec((BN, D), lambda i: (i, 0)),
           pl.BlockSpec((1, D), lambda i: (0, 0))]
    )
    return pl.pallas_call(
        _tc2_body,
        grid=(GRID,),
        in_specs=in_specs,
        out_specs=pl.BlockSpec((BN, D), lambda i: (i, 0)),
        out_shape=jax.ShapeDtypeStruct((N, D), jnp.float32),
    )(*accs, *degs, xl, bias2d)


def _pad_edges(e):
    src = jnp.concatenate(
        [e[0].astype(jnp.int32), jnp.zeros((E_PAD - E,), jnp.int32)])
    dst = jnp.concatenate(
        [e[1].astype(jnp.int32), jnp.full((E_PAD - E,), TRASH, jnp.int32)])
    return src.reshape(EROWS, IDXW), dst.reshape(EROWS, IDXW)


@jax.jit
def kernel(x, edge_index_r0, edge_index_r1, edge_index_r2, basis, coeff,
           loop_weight, h_bias):
    tc1_out = _tc1(x, basis, coeff, loop_weight)
    tables, xl = tc1_out[:R * NCH], tc1_out[R * NCH]

    edge_pairs = []
    for e in (edge_index_r0, edge_index_r1, edge_index_r2):
        s, d = _pad_edges(e)
        edge_pairs += [s, d]

    ones_h = jnp.ones((IDXW, CW), jnp.float32)
    zeros_h = jnp.zeros((ZROWS, CW), jnp.float32)
    sc_out = _sc(tables, edge_pairs, ones_h, zeros_h)
    accs, degs = sc_out[:R * NCH], sc_out[R * NCH:]

    return _tc2(accs, degs, xl, h_bias.reshape(1, D))


# speedup vs baseline: 2.5809x; 1.5257x over previous
import jax
import jax.numpy as jnp
from jax import lax
from jax.experimental import pallas as pl
from jax.experimental.pallas import tpu as pltpu
from jax.experimental.pallas import tpu_sc as plsc

N = 50000
E = 160000
D = 128
R = 3
NB = 2

CW = 32
NCH = D // CW
NSUB = 16
ROWS_PER_TILE = 3200
N_PAD = NSUB * ROWS_PER_TILE
TRASH = N_PAD - 1

IDXW = 128
K = 4
E_PAD = 163840
EROWS = E_PAD // IDXW
TILE_EROWS = EROWS // NSUB
GROUPS = TILE_EROWS // K
ZROWS = 128

BN = 2048
GRID = 25

ITEMS = (
    ('f', 0, 0, 0), ('f', 0, 1, 0), ('f', 0, 2, 1), ('f', 0, 3, 1),
    ('f', 1, 0, 0), ('f', 1, 1, 0), ('f', 1, 2, 1), ('f', 1, 3, 1),
    ('f', 2, 0, 0), ('f', 2, 1, 0), ('f', 2, 2, 1), ('f', 2, 3, 1),
    ('d', 0, 0, 0), ('d', 1, 1, 1), ('d', 2, 2, 1),
)


def _tc1_body(x_ref, basis_ref, coeff_ref, loop_ref, *out_refs):
    x = x_ref[...]
    y0 = jnp.dot(x, basis_ref[0], preferred_element_type=jnp.float32)
    y1 = jnp.dot(x, basis_ref[1], preferred_element_type=jnp.float32)
    for r in range(R):
        out_refs[r][...] = (y0 * coeff_ref[r:r + 1, 0:1]
                            + y1 * coeff_ref[r:r + 1, 1:2])
    out_refs[R][...] = jnp.dot(x, loop_ref[...],
                               preferred_element_type=jnp.float32)


def _tc1(x, basis, coeff, loop_weight):
    outs = [jax.ShapeDtypeStruct((N_PAD, D), jnp.float32) for _ in range(R + 1)]
    out_specs = [pl.BlockSpec((BN, D), lambda i: (i, 0)) for _ in range(R + 1)]
    return pl.pallas_call(
        _tc1_body,
        grid=(GRID,),
        in_specs=[
            pl.BlockSpec((BN, D), lambda i: (i, 0)),
            pl.BlockSpec((NB, D, D), lambda i: (0, 0, 0)),
            pl.BlockSpec((R, NB), lambda i: (0, 0)),
            pl.BlockSpec((D, D), lambda i: (0, 0)),
        ],
        out_specs=out_specs,
        out_shape=outs,
    )(x, basis, coeff, loop_weight)


def _sc_body(*refs):
    tables = refs[0:3]
    srcs = refs[3:15]
    dsts = refs[15:18]
    ones_hbm = refs[18]
    zeros_hbm = refs[19]
    outs = refs[20:24]
    acc, src_buf, dst_buf, rows_v, zbuf, ones_v, gsem = refs[24:31]

    core = lax.axis_index("c")
    tid = lax.axis_index("s")

    pltpu.sync_copy(zeros_hbm, zbuf)
    pltpu.sync_copy(ones_hbm, ones_v)

    def run_item(table, src_hbm, dst_hbm, out_ref, col0):
        def zero_body(j, carry):
            pltpu.sync_copy(
                zbuf, acc.at[pl.ds(tid * ROWS_PER_TILE + j * ZROWS, ZROWS)])
            return carry

        lax.fori_loop(0, ROWS_PER_TILE // ZROWS, zero_body, 0)
        plsc.subcore_barrier()

        def group_body(g, carry):
            base = tid * TILE_EROWS + g * K
            pltpu.sync_copy(dst_hbm.at[pl.ds(base, K)], dst_buf)
            if table is not None:
                pltpu.sync_copy(src_hbm.at[pl.ds(base, K)], src_buf)
                copies = [
                    pltpu.async_copy(table.at[src_buf.at[j]],
                                     rows_v.at[pl.ds(j * IDXW, IDXW)], gsem)
                    for j in range(K)
                ]
                for cp in copies:
                    cp.wait()
                for j in range(K):
                    pltpu.sync_copy(rows_v.at[pl.ds(j * IDXW, IDXW)],
                                    acc.at[dst_buf.at[j]], add=True)
            else:
                for j in range(K):
                    pltpu.sync_copy(ones_v, acc.at[dst_buf.at[j]], add=True)
            return carry

        lax.fori_loop(0, GROUPS, group_body, 0)
        plsc.subcore_barrier()
        pltpu.sync_copy(
            acc.at[pl.ds(tid * ROWS_PER_TILE, ROWS_PER_TILE)],
            out_ref.at[pl.ds(tid * ROWS_PER_TILE, ROWS_PER_TILE),
                       pl.ds(col0, CW)])

    for kind, r, c, owner in ITEMS:
        if kind == 'f':
            table = tables[r]
            src = srcs[r * NCH + c]
            out_ref = outs[r]
        else:
            table = None
            src = None
            out_ref = outs[R]

        @pl.when(core == owner)
        def _(table=table, src=src, r=r, c=c, out_ref=out_ref):
            run_item(table, src, dsts[r], out_ref, c * CW)


def _sc(tables, srcs, dsts, ones_h, zeros_h):
    mesh = plsc.VectorSubcoreMesh(core_axis_name="c", subcore_axis_name="s")
    out_type = [jax.ShapeDtypeStruct((N_PAD, D), jnp.float32)
                for _ in range(R + 1)]
    kern = pl.kernel(
        _sc_body,
        out_type=out_type,
        mesh=mesh,
        scratch_types=[
            pltpu.VMEM_SHARED((N_PAD, CW), jnp.float32),
            pltpu.VMEM((K, IDXW), jnp.int32),
            pltpu.VMEM((K, IDXW), jnp.int32),
            pltpu.VMEM((K * IDXW, CW), jnp.float32),
            pltpu.VMEM((ZROWS, CW), jnp.float32),
            pltpu.VMEM((IDXW, CW), jnp.float32),
            pltpu.SemaphoreType.DMA,
        ],
        compiler_params=pltpu.CompilerParams(use_tc_tiling_on_sc=False),
    )
    return kern(*tables, *srcs, *dsts, ones_h, zeros_h)


def _tc2_body(*refs):
    aggs = refs[0:3]
    deg_ref = refs[3]
    xl_ref = refs[4]
    bias_ref = refs[5]
    out_ref = refs[6]
    h = xl_ref[...] + bias_ref[...]
    deg = deg_ref[...]
    for r in range(R):
        inv = 1.0 / jnp.maximum(deg[:, r * CW:r * CW + 1], 1.0)
        h = h + aggs[r][...] * inv
    out_ref[...] = h


def _tc2(aggs, deg, xl, bias2d):
    in_specs = [pl.BlockSpec((BN, D), lambda i: (i, 0)) for _ in range(R + 2)]
    in_specs.append(pl.BlockSpec((1, D), lambda i: (0, 0)))
    return pl.pallas_call(
        _tc2_body,
        grid=(GRID,),
        in_specs=in_specs,
        out_specs=pl.BlockSpec((BN, D), lambda i: (i, 0)),
        out_shape=jax.ShapeDtypeStruct((N, D), jnp.float32),
    )(*aggs, deg, xl, bias2d)


def _pad_edges(e):
    src4 = e[0].astype(jnp.int32) * 4
    dst = jnp.concatenate(
        [e[1].astype(jnp.int32), jnp.full((E_PAD - E,), TRASH, jnp.int32)])
    srcs = []
    for c in range(NCH):
        s = jnp.concatenate([src4 + c, jnp.full((E_PAD - E,), c, jnp.int32)])
        srcs.append(s.reshape(EROWS, IDXW))
    return srcs, dst.reshape(EROWS, IDXW)


@jax.jit
def kernel(x, edge_index_r0, edge_index_r1, edge_index_r2, basis, coeff,
           loop_weight, h_bias):
    tc1_out = _tc1(x, basis, coeff, loop_weight)
    tables = [t.reshape(4 * N_PAD, CW) for t in tc1_out[:R]]
    xl = tc1_out[R]

    srcs, dsts = [], []
    for e in (edge_index_r0, edge_index_r1, edge_index_r2):
        s4, d = _pad_edges(e)
        srcs += s4
        dsts.append(d)

    ones_h = jnp.ones((IDXW, CW), jnp.float32)
    zeros_h = jnp.zeros((ZROWS, CW), jnp.float32)
    sc_out = _sc(tables, srcs, dsts, ones_h, zeros_h)
    aggs, deg = sc_out[:R], sc_out[R]

    return _tc2(aggs, deg, xl, h_bias.reshape(1, D))


# speedup vs baseline: 2.9599x; 1.1468x over previous
import jax
import jax.numpy as jnp
from jax import lax
from jax.experimental import pallas as pl
from jax.experimental.pallas import tpu as pltpu
from jax.experimental.pallas import tpu_sc as plsc

N = 50000
E = 160000
D = 128
R = 3
NB = 2

CW = 32
NCH = D // CW
NSUB = 16
ROWS_PER_TILE = 3200
N_PAD = NSUB * ROWS_PER_TILE
TRASH = N_PAD - 1

IDXW = 128
PK = 2
SG = 16
E_PAD = 163840
EROWS = E_PAD // IDXW
TILE_EROWS = EROWS // NSUB
NSG = TILE_EROWS // SG
PAIRS = SG // (2 * PK)
ZROWS = 128

BN = 2048
GRID = 25

ITEMS = (
    ('f', 0, 0, 0), ('f', 0, 1, 0), ('f', 0, 2, 1), ('f', 0, 3, 1),
    ('f', 1, 0, 0), ('f', 1, 1, 0), ('f', 1, 2, 1), ('f', 1, 3, 1),
    ('f', 2, 0, 0), ('f', 2, 1, 0), ('f', 2, 2, 1), ('f', 2, 3, 1),
    ('d', 0, 0, 0), ('d', 1, 1, 1), ('d', 2, 2, 1),
)


def _tc1_body(x_ref, basis_ref, coeff_ref, loop_ref, *out_refs):
    x = x_ref[...]
    y0 = jnp.dot(x, basis_ref[0], preferred_element_type=jnp.float32)
    y1 = jnp.dot(x, basis_ref[1], preferred_element_type=jnp.float32)
    for r in range(R):
        out_refs[r][...] = (y0 * coeff_ref[r:r + 1, 0:1]
                            + y1 * coeff_ref[r:r + 1, 1:2])
    out_refs[R][...] = jnp.dot(x, loop_ref[...],
                               preferred_element_type=jnp.float32)


def _tc1(x, basis, coeff, loop_weight):
    outs = [jax.ShapeDtypeStruct((N_PAD, D), jnp.float32) for _ in range(R + 1)]
    out_specs = [pl.BlockSpec((BN, D), lambda i: (i, 0)) for _ in range(R + 1)]
    return pl.pallas_call(
        _tc1_body,
        grid=(GRID,),
        in_specs=[
            pl.BlockSpec((BN, D), lambda i: (i, 0)),
            pl.BlockSpec((NB, D, D), lambda i: (0, 0, 0)),
            pl.BlockSpec((R, NB), lambda i: (0, 0)),
            pl.BlockSpec((D, D), lambda i: (0, 0)),
        ],
        out_specs=out_specs,
        out_shape=outs,
    )(x, basis, coeff, loop_weight)


def _sc_body(*refs):
    tables = refs[0:3]
    srcs = refs[3:15]
    dsts = refs[15:18]
    ones_hbm = refs[18]
    zeros_hbm = refs[19]
    outs = refs[20:24]
    (acc, src_sg, dst_sg, rows_a, rows_b, zbuf, ones_v,
     gsem_a, gsem_b) = refs[24:33]

    core = lax.axis_index("c")
    tid = lax.axis_index("s")

    pltpu.sync_copy(zeros_hbm, zbuf)
    pltpu.sync_copy(ones_hbm, ones_v)

    def run_item(table, src_hbm, dst_hbm, out_ref, col0):
        def zero_body(j, carry):
            pltpu.sync_copy(
                zbuf, acc.at[pl.ds(tid * ROWS_PER_TILE + j * ZROWS, ZROWS)])
            return carry

        lax.fori_loop(0, ROWS_PER_TILE // ZROWS, zero_body, 0)
        plsc.subcore_barrier()

        if table is not None:
            def fire(buf, sem, row0):
                for j in range(PK):
                    pltpu.async_copy(table.at[src_sg.at[row0 + j]],
                                     buf.at[pl.ds(j * IDXW, IDXW)], sem)

            def drain_scatter(buf, sem, row0):
                for j in range(PK):
                    pltpu.make_async_copy(
                        table.at[src_sg.at[row0 + j]],
                        buf.at[pl.ds(j * IDXW, IDXW)], sem).wait()
                for j in range(PK):
                    pltpu.sync_copy(buf.at[pl.ds(j * IDXW, IDXW)],
                                    acc.at[dst_sg.at[row0 + j]], add=True)

            def sg_body(s, carry):
                base = tid * TILE_EROWS + s * SG
                pltpu.sync_copy(src_hbm.at[pl.ds(base, SG)], src_sg)
                pltpu.sync_copy(dst_hbm.at[pl.ds(base, SG)], dst_sg)
                fire(rows_a, gsem_a, 0)

                def pair_body(i, carry2):
                    row_a = 2 * PK * i
                    fire(rows_b, gsem_b, row_a + PK)
                    drain_scatter(rows_a, gsem_a, row_a)

                    @pl.when(i < PAIRS - 1)
                    def _():
                        fire(rows_a, gsem_a, row_a + 2 * PK)

                    drain_scatter(rows_b, gsem_b, row_a + PK)
                    return carry2

                lax.fori_loop(0, PAIRS, pair_body, 0)
                return carry

            lax.fori_loop(0, NSG, sg_body, 0)
        else:
            def sg_body_d(s, carry):
                base = tid * TILE_EROWS + s * SG
                pltpu.sync_copy(dst_hbm.at[pl.ds(base, SG)], dst_sg)

                def row_body(j, carry2):
                    pltpu.sync_copy(ones_v, acc.at[dst_sg.at[j]], add=True)
                    return carry2

                lax.fori_loop(0, SG, row_body, 0)
                return carry

            lax.fori_loop(0, NSG, sg_body_d, 0)

        plsc.subcore_barrier()
        pltpu.sync_copy(
            acc.at[pl.ds(tid * ROWS_PER_TILE, ROWS_PER_TILE)],
            out_ref.at[pl.ds(tid * ROWS_PER_TILE, ROWS_PER_TILE),
                       pl.ds(col0, CW)])

    for kind, r, c, owner in ITEMS:
        if kind == 'f':
            table = tables[r]
            src = srcs[r * NCH + c]
            out_ref = outs[r]
        else:
            table = None
            src = None
            out_ref = outs[R]

        @pl.when(core == owner)
        def _(table=table, src=src, r=r, c=c, out_ref=out_ref):
            run_item(table, src, dsts[r], out_ref, c * CW)


def _sc(tables, srcs, dsts, ones_h, zeros_h):
    mesh = plsc.VectorSubcoreMesh(core_axis_name="c", subcore_axis_name="s")
    out_type = [jax.ShapeDtypeStruct((N_PAD, D), jnp.float32)
                for _ in range(R + 1)]
    kern = pl.kernel(
        _sc_body,
        out_type=out_type,
        mesh=mesh,
        scratch_types=[
            pltpu.VMEM_SHARED((N_PAD, CW), jnp.float32),
            pltpu.VMEM((SG, IDXW), jnp.int32),
            pltpu.VMEM((SG, IDXW), jnp.int32),
            pltpu.VMEM((PK * IDXW, CW), jnp.float32),
            pltpu.VMEM((PK * IDXW, CW), jnp.float32),
            pltpu.VMEM((ZROWS, CW), jnp.float32),
            pltpu.VMEM((IDXW, CW), jnp.float32),
            pltpu.SemaphoreType.DMA,
            pltpu.SemaphoreType.DMA,
        ],
        compiler_params=pltpu.CompilerParams(use_tc_tiling_on_sc=False),
    )
    return kern(*tables, *srcs, *dsts, ones_h, zeros_h)


def _tc2_body(*refs):
    aggs = refs[0:3]
    deg_ref = refs[3]
    xl_ref = refs[4]
    bias_ref = refs[5]
    out_ref = refs[6]
    h = xl_ref[...] + bias_ref[...]
    deg = deg_ref[...]
    for r in range(R):
        inv = 1.0 / jnp.maximum(deg[:, r * CW:r * CW + 1], 1.0)
        h = h + aggs[r][...] * inv
    out_ref[...] = h


def _tc2(aggs, deg, xl, bias2d):
    in_specs = [pl.BlockSpec((BN, D), lambda i: (i, 0)) for _ in range(R + 2)]
    in_specs.append(pl.BlockSpec((1, D), lambda i: (0, 0)))
    return pl.pallas_call(
        _tc2_body,
        grid=(GRID,),
        in_specs=in_specs,
        out_specs=pl.BlockSpec((BN, D), lambda i: (i, 0)),
        out_shape=jax.ShapeDtypeStruct((N, D), jnp.float32),
    )(*aggs, deg, xl, bias2d)


def _pad_edges(e):
    src4 = e[0].astype(jnp.int32) * 4
    dst = jnp.concatenate(
        [e[1].astype(jnp.int32), jnp.full((E_PAD - E,), TRASH, jnp.int32)])
    srcs = []
    for c in range(NCH):
        s = jnp.concatenate([src4 + c, jnp.full((E_PAD - E,), c, jnp.int32)])
        srcs.append(s.reshape(EROWS, IDXW))
    return srcs, dst.reshape(EROWS, IDXW)


@jax.jit
def kernel(x, edge_index_r0, edge_index_r1, edge_index_r2, basis, coeff,
           loop_weight, h_bias):
    tc1_out = _tc1(x, basis, coeff, loop_weight)
    tables = [t.reshape(4 * N_PAD, CW) for t in tc1_out[:R]]
    xl = tc1_out[R]

    srcs, dsts = [], []
    for e in (edge_index_r0, edge_index_r1, edge_index_r2):
        s4, d = _pad_edges(e)
        srcs += s4
        dsts.append(d)

    ones_h = jnp.ones((IDXW, CW), jnp.float32)
    zeros_h = jnp.zeros((ZROWS, CW), jnp.float32)
    sc_out = _sc(tables, srcs, dsts, ones_h, zeros_h)
    aggs, deg = sc_out[:R], sc_out[R]

    return _tc2(aggs, deg, xl, h_bias.reshape(1, D))
